# baseline (speedup 1.0000x reference)
import functools

import jax
import jax.numpy as jnp
from jax import lax
from jax.experimental import pallas as pl
from jax.experimental.pallas import tpu as pltpu
from jax.experimental.pallas import tpu_sc as plsc

G0, G1, NH = 480, 360, 32
NPTS = 100000
NCELL = G1 * G1
NCP = 131072
NSEG = G0 * G1
BLK = 2000
NBLK = NPTS // BLK
EPS = 1e-5
NEG = -1e30


def _stats_block(h):
    s = jnp.sum(h, axis=0)[None, None, :]
    sq = jnp.sum(h * h, axis=0)[None, None, :]
    return s, sq


def _p0_kernel(x_ref, s_ref, sq_ref):
    @pl.when(pl.program_id(1) == 0)
    def _():
        s_ref[...] = jnp.zeros_like(s_ref)
        sq_ref[...] = jnp.zeros_like(sq_ref)
    x = x_ref[0]
    s, sq = _stats_block(x)
    s_ref[...] += s
    sq_ref[...] += sq


def _layers(x, args, n):
    sc0, sh0 = args[0][0, 0], args[1][0, 0]
    h = x * sc0 + sh0
    for i in range(1, n + 1):
        W = args[2 + 4 * (i - 1)]
        b = args[3 + 4 * (i - 1)][0, 0]
        h = jnp.dot(h.astype(jnp.bfloat16), W.astype(jnp.bfloat16),
                    preferred_element_type=jnp.float32) + b
        if i < n:
            sc = args[4 + 4 * (i - 1)][0, 0]
            sh = args[5 + 4 * (i - 1)][0, 0]
            h = jnp.maximum(h * sc + sh, 0.0)
    return h


def _stat_pass_kernel(n, *refs):
    x_ref = refs[0]
    args = [r[...] if r.shape[0] == 1 and len(r.shape) == 3 else r[...]
            for r in refs[1:-2]]
    s_ref, sq_ref = refs[-2], refs[-1]

    @pl.when(pl.program_id(1) == 0)
    def _():
        s_ref[...] = jnp.zeros_like(s_ref)
        sq_ref[...] = jnp.zeros_like(sq_ref)

    h = _layers(x_ref[0], args, n)
    s, sq = _stats_block(h)
    s_ref[...] += s
    sq_ref[...] += sq


def _p4_kernel(*refs):
    x_ref = refs[0]
    args = [r[...] for r in refs[1:-1]]
    out_ref = refs[-1]
    h = _layers(x_ref[0], args, 4)
    for ci in range(4):
        out_ref[ci] = h[:, ci * 128:(ci + 1) * 128]


def _vec(v):
    return v.reshape(1, 1, -1)


def _moment_specs(c):
    blk = pl.BlockSpec((1, 1, c), lambda b, i: (b, 0, 0))
    return [blk, blk]


def _vspec(c):
    return pl.BlockSpec((1, 1, c), lambda b, i: (0, 0, 0))


def _wspec(shape):
    return pl.BlockSpec(shape, lambda b, i: (0,) * len(shape))


def _scale_shift(s, sq, g, b):
    mean = s[0] / NPTS
    var = sq[0] / NPTS - mean * mean
    scale = g * lax.rsqrt(var + EPS)
    shift = b - mean * scale
    return scale, shift


def _mlp(xp, params):
    (bn0_g, bn0_b, W1, b1, bn1_g, bn1_b, W2, b2, bn2_g, bn2_b,
     W3, b3, bn3_g, bn3_b, W4, b4) = params
    grid = (2, NBLK)
    xspec = pl.BlockSpec((1, BLK, 8), lambda b, i: (b, i, 0))
    cp = pltpu.CompilerParams(
        dimension_semantics=("arbitrary", "arbitrary"))
    mom = lambda c: [jax.ShapeDtypeStruct((2, 1, c), jnp.float32)] * 2

    s0, sq0 = pl.pallas_call(
        _p0_kernel, grid=grid,
        in_specs=[xspec], out_specs=_moment_specs(8),
        out_shape=mom(8), compiler_params=cp)(xp)
    results = []
    for b in range(2):
        sc0, sh0 = _scale_shift(s0[b], sq0[b], bn0_g, bn0_b)
        args = [_vec(sc0), _vec(sh0), W1, _vec(b1)]
        specs = [_vspec(8), _vspec(8), _wspec(W1.shape), _vspec(64)]
        dims = [64, 128, 256]
        gs = [bn1_g, bn2_g, bn3_g]
        bs = [bn1_b, bn2_b, bn3_b]
        Ws = [W2, W3, W4]
        lbs = [b2, b3, b4]
        xb = xp[b:b + 1]
        gridb = (1, NBLK)
        for n in (1, 2, 3):
            c = dims[n - 1]
            s, sq = pl.pallas_call(
                functools.partial(_stat_pass_kernel, n), grid=gridb,
                in_specs=[xspec] + specs,
                out_specs=_moment_specs(c),
                out_shape=mom(c)[0:1] + mom(c)[0:1],
                compiler_params=cp)(xb, *args)
            sc, sh = _scale_shift(s[0], sq[0], gs[n - 1], bs[n - 1])
            W, lb = Ws[n - 1], lbs[n - 1]
            args += [_vec(sc), _vec(sh), W, _vec(lb)]
            specs += [_vspec(c), _vspec(c), _wspec(W.shape),
                      _vspec(W.shape[1])]
        h4 = pl.pallas_call(
            _p4_kernel, grid=gridb,
            in_specs=[xspec] + specs,
            out_specs=pl.BlockSpec((4, BLK, 128), lambda b, i: (0, i, 0)),
            out_shape=jax.ShapeDtypeStruct((4, NPTS, 128), jnp.float32),
            compiler_params=cp)(xb, *args)
        results.append(h4)
    return results


NW = 32
CPW = NCP // NW
NSUB = 8
CPS = CPW // NSUB
PTS_PAD = 102400
CHUNK = 2048
NCHUNK = PTS_PAD // CHUNK
SUBCAP = PTS_PAD + CHUNK
MCH = 384
B1B_CH = 512
B1B_STG = 1024
NCI = 4


def _scatter_body(cells_hbm, h4v_hbm, grid_hbm, lists_hbm, subs_hbm,
                  cbuf, lbuf, ibuf, rbuf, acc,
                  st0, st1, st2, st3, st4, st5, st6, st7, sema, semw):
    stg = [st0, st1, st2, st3, st4, st5, st6, st7]
    wid = lax.axis_index("s") * 2 + lax.axis_index("c")
    base_cell = wid * CPW
    lane = lax.iota(jnp.int32, 16)

    def chunk_body(c, carry):
        cursor, flushed = carry
        pltpu.sync_copy(cells_hbm.at[pl.ds(c * CHUNK, CHUNK)], cbuf)

        def vec_body(i, cur):
            v = cbuf[pl.ds(i * 16, 16)]
            local = v - base_cell
            m = (local >= 0) & (local < CPW)
            mi = m.astype(jnp.int32)
            pid = c * CHUNK + i * 16 + lane
            packed = pid * CPW + local
            off = cur + plsc.cumsum(mi) - mi
            plsc.store_scatter(lbuf, [off], packed, mask=m)
            return cur + jnp.sum(mi)

        cursor = lax.fori_loop(0, CHUNK // 16, vec_body, cursor)

        def do_flush(cf):
            cur, fl = cf
            pltpu.sync_copy(lbuf.at[pl.ds(0, CHUNK)],
                            lists_hbm.at[pl.ds(pl.multiple_of(
                                wid * PTS_PAD + fl, CHUNK), CHUNK)])

            def mv(i, _):
                lbuf[pl.ds(i * 16, 16)] = lbuf[pl.ds(CHUNK + i * 16, 16)]
                return 0

            lax.fori_loop(0, CHUNK // 16, mv, 0)
            return cur - CHUNK, fl + CHUNK

        return lax.cond(cursor >= CHUNK, do_flush, lambda cf: cf,
                        (cursor, flushed))

    cursor, flushed = lax.fori_loop(0, NCHUNK, chunk_body,
                                    (jnp.int32(0), jnp.int32(0)))

    @pl.when(cursor > 0)
    def _():
        pltpu.sync_copy(lbuf.at[pl.ds(0, CHUNK)],
                        lists_hbm.at[pl.ds(pl.multiple_of(
                            wid * PTS_PAD + flushed, CHUNK), CHUNK)])

    count = flushed + cursor

    sub_hbase = wid * (NSUB * SUBCAP)
    nch1 = lax.div(count + B1B_CH - 1, B1B_CH)

    def p1b_chunk(ci, carry):
        pltpu.sync_copy(lists_hbm.at[pl.ds(pl.multiple_of(
            wid * PTS_PAD + ci * B1B_CH, B1B_CH), B1B_CH)],
            lbuf.at[pl.ds(0, B1B_CH)])
        n_c = jnp.minimum(count - ci * B1B_CH, B1B_CH)
        nvec = lax.div(n_c + 15, 16)

        def vec_body(i, cy):
            w16 = lbuf[pl.ds(i * 16, 16)]
            local = jnp.bitwise_and(w16, CPW - 1)
            sub = lax.shift_right_logical(local, 9)
            valid = (i * 16 + lane) < n_c
            out = []
            for s in range(NSUB):
                ms = valid & (sub == s)
                mi = ms.astype(jnp.int32)
                off = cy[s] + plsc.cumsum(mi) - mi
                plsc.store_scatter(stg[s], [off], w16, mask=ms)
                out.append(cy[s] + jnp.sum(mi))
            return tuple(out) + cy[NSUB:]

        cy = lax.fori_loop(0, nvec, vec_body, carry)

        out = list(cy)
        for s in range(NSUB):
            def do_flush(cf, s=s):
                cur, fl = cf
                pltpu.sync_copy(
                    stg[s].at[pl.ds(0, B1B_CH)],
                    subs_hbm.at[pl.ds(pl.multiple_of(
                        sub_hbase + s * SUBCAP + fl, B1B_CH), B1B_CH)])

                def mv(i, _, s=s):
                    stg[s][pl.ds(i * 16, 16)] = \
                        stg[s][pl.ds(B1B_CH + i * 16, 16)]
                    return 0

                lax.fori_loop(0, B1B_CH // 16, mv, 0)
                return cur - B1B_CH, fl + B1B_CH

            cur, fl = lax.cond(out[s] >= B1B_CH, do_flush,
                               lambda cf: cf, (out[s], out[NSUB + s]))
            out[s], out[NSUB + s] = cur, fl
        return tuple(out)

    z = jnp.int32(0)
    carry = lax.fori_loop(0, nch1, p1b_chunk, (z,) * (2 * NSUB))
    sub_counts = []
    for s in range(NSUB):
        scur, sfl = carry[s], carry[NSUB + s]

        @pl.when(scur > 0)
        def _(s=s, sfl=sfl):
            pltpu.sync_copy(
                stg[s].at[pl.ds(0, B1B_CH)],
                subs_hbm.at[pl.ds(pl.multiple_of(
                    sub_hbase + s * SUBCAP + sfl, B1B_CH), B1B_CH)])

        sub_counts.append(scur + sfl)

    for s in range(NSUB):
        plsc.store_scatter(cbuf, [jnp.full((16,), s, jnp.int32)],
                           jnp.full((16,), sub_counts[s], jnp.int32),
                           mask=(lane == 0))

    neg = jnp.full((16,), NEG, jnp.float32)
    cntv = cbuf[pl.ds(0, 16)]

    def tile_body(t, _):
        ci = lax.shift_right_logical(t, 3)
        s = jnp.bitwise_and(t, NSUB - 1)
        cbase = base_cell + s * CPS
        cnt = lax.gather(
            cntv, jnp.full((16, 1), s, jnp.int32),
            lax.GatherDimensionNumbers(
                offset_dims=(), collapsed_slice_dims=(0,),
                start_index_map=(0,)),
            slice_sizes=(1,),
            mode=lax.GatherScatterMode.PROMISE_IN_BOUNDS)[0]
        nmc = lax.div(cnt + MCH - 1, MCH)

        def prep_issue(mc):
            pltpu.sync_copy(
                subs_hbm.at[pl.ds(pl.multiple_of(
                    sub_hbase + s * SUBCAP + mc * MCH, 128), MCH)],
                lbuf.at[pl.ds(0, MCH)])

            def idx_body(i, _):
                w16 = lbuf[pl.ds(i * 16, 16)]
                pid = lax.shift_right_logical(w16, 12)
                pid = jnp.clip(pid, 0, NPTS - 1)
                ibuf[pl.ds(i * 16, 16)] = ci * NPTS + pid
                return 0

            lax.fori_loop(0, MCH // 16, idx_body, 0)
            pltpu.async_copy(h4v_hbm.at[ibuf], rbuf, sema)

        @pl.when(nmc > 0)
        def _():
            prep_issue(0)

        @pl.when(t > 0)
        def _():
            pltpu.make_async_copy(
                acc, grid_hbm.at[0, pl.ds(0, CPS)], semw).wait()

        def init_body(i, _):
            for f in range(8):
                acc[i, pl.ds(f * 16, 16)] = neg
            return 0

        lax.fori_loop(0, CPS, init_body, 0)

        def mc_body(mc, _):
            pltpu.make_async_copy(h4v_hbm.at[ibuf], rbuf, sema).wait()
            n_c = jnp.minimum(cnt - mc * MCH, MCH)

            def rmw(j, cl):
                for f in range(8):
                    row = rbuf[j, pl.ds(f * 16, 16)]
                    a = acc[cl, pl.ds(f * 16, 16)]
                    acc[cl, pl.ds(f * 16, 16)] = jnp.maximum(a, row)

            nfull = lax.div(n_c, 16)

            def mg(i, _):
                wv = lbuf[pl.ds(i * 16, 16)]
                for k in range(16):
                    rmw(i * 16 + k, jnp.bitwise_and(wv[k], CPS - 1))
                return 0

            lax.fori_loop(0, nfull, mg, 0)

            tbase = nfull * 16
            wv = lbuf[pl.ds(tbase, 16)]
            for k in range(16):
                @pl.when(tbase + k < n_c)
                def _(k=k):
                    rmw(tbase + k, jnp.bitwise_and(wv[k], CPS - 1))

            @pl.when(mc + 1 < nmc)
            def _():
                prep_issue(mc + 1)

            return 0

        lax.fori_loop(0, nmc, mc_body, 0)

        pltpu.async_copy(
            acc, grid_hbm.at[ci, pl.ds(pl.multiple_of(cbase, CPS), CPS)],
            semw)
        return 0

    lax.fori_loop(0, NCI * NSUB, tile_body, 0)
    pltpu.make_async_copy(acc, grid_hbm.at[0, pl.ds(0, CPS)], semw).wait()


def _scatter_max(cells_b, h4_b):
    h4v = h4_b.reshape(NCI * NPTS, 128)
    mesh = plsc.VectorSubcoreMesh(core_axis_name="c", subcore_axis_name="s")
    grid_out, _, _ = pl.kernel(
        _scatter_body,
        out_type=(jax.ShapeDtypeStruct((NCI, NCP, 128), jnp.float32),
                  jax.ShapeDtypeStruct((NW * PTS_PAD,), jnp.int32),
                  jax.ShapeDtypeStruct((NW * NSUB * SUBCAP,), jnp.int32)),
        mesh=mesh,
        compiler_params=pltpu.CompilerParams(needs_layout_passes=False),
        scratch_types=(
            [pltpu.VMEM((CHUNK,), jnp.int32),
             pltpu.VMEM((2 * CHUNK,), jnp.int32),
             pltpu.VMEM((MCH,), jnp.int32),
             pltpu.VMEM((MCH, 128), jnp.float32),
             pltpu.VMEM((CPS, 128), jnp.float32)]
            + [pltpu.VMEM((B1B_STG,), jnp.int32)] * NSUB
            + [pltpu.SemaphoreType.DMA,
               pltpu.SemaphoreType.DMA]),
    )(cells_b, h4v)
    return grid_out


CB = 2048


def _final_kernel(g_ref, wc_ref, bc_ref, out_ref):
    g = jnp.concatenate([g_ref[0, ci] for ci in range(4)],
                        axis=1)
    present = (g[:, 0:1] > NEG)
    gm = jnp.where(present, g, 0.0)
    nhT = lax.dot_general(wc_ref[...], gm, (((0,), (1,)), ((), ())),
                          preferred_element_type=jnp.float32)
    nhT = jnp.maximum(nhT + bc_ref[0, 0][:, None], 0.0)
    out_ref[0] = jnp.where(present.T, nhT, 0.0)


def _final(grid2, Wc, bc):
    nb = NCP // CB
    return pl.pallas_call(
        _final_kernel, grid=(2, nb),
        in_specs=[
            pl.BlockSpec((1, NCI, CB, 128), lambda b, i: (b, 0, i, 0)),
            pl.BlockSpec((512, NH), lambda b, i: (0, 0)),
            pl.BlockSpec((1, 1, NH), lambda b, i: (0, 0, 0)),
        ],
        out_specs=pl.BlockSpec((1, NH, CB), lambda b, i: (b, 0, i)),
        out_shape=jax.ShapeDtypeStruct((2, NH, NCP), jnp.float32),
        compiler_params=pltpu.CompilerParams(
            dimension_semantics=("arbitrary", "arbitrary")),
    )(grid2, Wc, _vec(bc))


def kernel(pt_fea, xy_ind, circular_padding, bn0_g, bn0_b, W1, b1, bn1_g,
           bn1_b, W2, b2, bn2_g, bn2_b, W3, b3, bn3_g, bn3_b, W4, b4,
           Wc, bc):
    del circular_padding
    xp = jnp.pad(pt_fea, ((0, 0), (0, 0), (0, 1)))
    W1p = jnp.pad(W1, ((0, 1), (0, 0)))
    g0p = jnp.pad(bn0_g, (0, 1))
    b0p = jnp.pad(bn0_b, (0, 1))
    params = (g0p, b0p, W1p, b1, bn1_g, bn1_b, W2, b2, bn2_g, bn2_b,
              W3, b3, bn3_g, bn3_b, W4, b4)
    h4 = _mlp(xp, params)
    cells = xy_ind[..., 0] * G1 + xy_ind[..., 1]
    cells_p = jnp.pad(cells, ((0, 0), (0, PTS_PAD - NPTS)),
                      constant_values=NCP - 1)
    grid2 = jnp.stack([_scatter_max(cells_p[b], h4[b]) for b in range(2)])

    outc = _final(grid2, Wc, bc)[:, :, :NCELL]
    out = jnp.concatenate(
        [outc, jnp.zeros((2, NH, NSEG - NCELL), jnp.float32)], axis=2)
    return out.reshape(2, NH, G0, G1)


# speedup vs baseline: 2.9488x; 2.9488x over previous
import functools

import jax
import jax.numpy as jnp
from jax import lax
from jax.experimental import pallas as pl
from jax.experimental.pallas import tpu as pltpu
from jax.experimental.pallas import tpu_sc as plsc

G0, G1, NH = 480, 360, 32
NPTS = 100000
NCELL = G1 * G1
NCP = 131072
NSEG = G0 * G1
BLK = 2000
NBLK = NPTS // BLK
EPS = 1e-5
NEG = -1e30


def _stats_block(h):
    s = jnp.sum(h, axis=0)[None, None, :]
    sq = jnp.sum(h * h, axis=0)[None, None, :]
    return s, sq


def _p0_kernel(x_ref, s_ref, sq_ref):
    @pl.when(pl.program_id(1) == 0)
    def _():
        s_ref[...] = jnp.zeros_like(s_ref)
        sq_ref[...] = jnp.zeros_like(sq_ref)
    x = x_ref[0]
    s, sq = _stats_block(x)
    s_ref[...] += s
    sq_ref[...] += sq


def _layers(x, args, n):
    sc0, sh0 = args[0][0, 0], args[1][0, 0]
    h = x * sc0 + sh0
    for i in range(1, n + 1):
        W = args[2 + 4 * (i - 1)]
        b = args[3 + 4 * (i - 1)][0, 0]
        h = jnp.dot(h, W, preferred_element_type=jnp.float32) + b
        if i < n:
            sc = args[4 + 4 * (i - 1)][0, 0]
            sh = args[5 + 4 * (i - 1)][0, 0]
            h = jnp.maximum(h * sc + sh, 0.0)
    return h


def _stat_pass_kernel(n, *refs):
    x_ref = refs[0]
    args = [r[...] if r.shape[0] == 1 and len(r.shape) == 3 else r[...]
            for r in refs[1:-2]]
    s_ref, sq_ref = refs[-2], refs[-1]

    @pl.when(pl.program_id(1) == 0)
    def _():
        s_ref[...] = jnp.zeros_like(s_ref)
        sq_ref[...] = jnp.zeros_like(sq_ref)

    h = _layers(x_ref[0], args, n)
    s, sq = _stats_block(h)
    s_ref[...] += s
    sq_ref[...] += sq


def _p4_kernel(*refs):
    x_ref = refs[0]
    args = [r[...] for r in refs[1:-1]]
    out_ref = refs[-1]
    h = _layers(x_ref[0], args, 4)
    for ci in range(4):
        out_ref[ci] = h[:, ci * 128:(ci + 1) * 128]


def _vec(v):
    return v.reshape(1, 1, -1)


def _moment_specs(c):
    blk = pl.BlockSpec((1, 1, c), lambda b, i: (b, 0, 0))
    return [blk, blk]


def _vspec(c):
    return pl.BlockSpec((1, 1, c), lambda b, i: (0, 0, 0))


def _wspec(shape):
    return pl.BlockSpec(shape, lambda b, i: (0,) * len(shape))


def _scale_shift(s, sq, g, b):
    mean = s[0] / NPTS
    var = sq[0] / NPTS - mean * mean
    scale = g * lax.rsqrt(var + EPS)
    shift = b - mean * scale
    return scale, shift


def _mlp(xp, params):
    (bn0_g, bn0_b, W1, b1, bn1_g, bn1_b, W2, b2, bn2_g, bn2_b,
     W3, b3, bn3_g, bn3_b, W4, b4) = params
    grid = (2, NBLK)
    xspec = pl.BlockSpec((1, BLK, 8), lambda b, i: (b, i, 0))
    cp = pltpu.CompilerParams(
        dimension_semantics=("arbitrary", "arbitrary"))
    mom = lambda c: [jax.ShapeDtypeStruct((2, 1, c), jnp.float32)] * 2

    s0, sq0 = pl.pallas_call(
        _p0_kernel, grid=grid,
        in_specs=[xspec], out_specs=_moment_specs(8),
        out_shape=mom(8), compiler_params=cp)(xp)
    results = []
    for b in range(2):
        sc0, sh0 = _scale_shift(s0[b], sq0[b], bn0_g, bn0_b)
        args = [_vec(sc0), _vec(sh0), W1, _vec(b1)]
        specs = [_vspec(8), _vspec(8), _wspec(W1.shape), _vspec(64)]
        dims = [64, 128, 256]
        gs = [bn1_g, bn2_g, bn3_g]
        bs = [bn1_b, bn2_b, bn3_b]
        Ws = [W2, W3, W4]
        lbs = [b2, b3, b4]
        xb = xp[b:b + 1]
        gridb = (1, NBLK)
        for n in (1, 2, 3):
            c = dims[n - 1]
            s, sq = pl.pallas_call(
                functools.partial(_stat_pass_kernel, n), grid=gridb,
                in_specs=[xspec] + specs,
                out_specs=_moment_specs(c),
                out_shape=mom(c)[0:1] + mom(c)[0:1],
                compiler_params=cp)(xb, *args)
            sc, sh = _scale_shift(s[0], sq[0], gs[n - 1], bs[n - 1])
            W, lb = Ws[n - 1], lbs[n - 1]
            args += [_vec(sc), _vec(sh), W, _vec(lb)]
            specs += [_vspec(c), _vspec(c), _wspec(W.shape),
                      _vspec(W.shape[1])]
        h4 = pl.pallas_call(
            _p4_kernel, grid=gridb,
            in_specs=[xspec] + specs,
            out_specs=pl.BlockSpec((4, BLK, 128), lambda b, i: (0, i, 0)),
            out_shape=jax.ShapeDtypeStruct((4, NPTS, 128), jnp.float32),
            compiler_params=cp)(xb, *args)
        results.append(h4)
    return results


NW = 32
CPW = NCP // NW
NSUB = 8
CPS = CPW // NSUB
PTS_PAD = 102400
CHUNK = 2048
NCHUNK = PTS_PAD // CHUNK
SUBCAP = PTS_PAD + CHUNK
MCH = 64
B1B_CH = 1024
B1B_STG = 2048
NCI = 4


def _scatter_body(cells_hbm, h4v_hbm, grid_hbm, lists_hbm, subs_hbm,
                  cbuf, lbuf, ibuf0, ibuf1, ibuf2, ibuf3,
                  rbuf0, rbuf1, rbuf2, rbuf3, acc,
                  st0, st1, st2, st3, st4, st5, st6, st7,
                  sem0, sem1, sem2, sem3):
    stg = [st0, st1, st2, st3, st4, st5, st6, st7]
    wid = lax.axis_index("s") * 2 + lax.axis_index("c")
    base_cell = wid * CPW
    lane = lax.iota(jnp.int32, 16)

    def chunk_body(c, carry):
        cursor, flushed = carry
        pltpu.sync_copy(cells_hbm.at[pl.ds(c * CHUNK, CHUNK)], cbuf)

        def vec_body(i, cur):
            v = cbuf[pl.ds(i * 16, 16)]
            local = v - base_cell
            m = (local >= 0) & (local < CPW)
            mi = m.astype(jnp.int32)
            pid = c * CHUNK + i * 16 + lane
            packed = pid * CPW + local
            off = cur + plsc.cumsum(mi) - mi
            plsc.store_scatter(lbuf, [off], packed, mask=m)
            return cur + jnp.sum(mi)

        cursor = lax.fori_loop(0, CHUNK // 16, vec_body, cursor)

        def do_flush(cf):
            cur, fl = cf
            pltpu.sync_copy(lbuf.at[pl.ds(0, CHUNK)],
                            lists_hbm.at[pl.ds(pl.multiple_of(
                                wid * PTS_PAD + fl, CHUNK), CHUNK)])

            def mv(i, _):
                lbuf[pl.ds(i * 16, 16)] = lbuf[pl.ds(CHUNK + i * 16, 16)]
                return 0

            lax.fori_loop(0, CHUNK // 16, mv, 0)
            return cur - CHUNK, fl + CHUNK

        return lax.cond(cursor >= CHUNK, do_flush, lambda cf: cf,
                        (cursor, flushed))

    cursor, flushed = lax.fori_loop(0, NCHUNK, chunk_body,
                                    (jnp.int32(0), jnp.int32(0)))

    @pl.when(cursor > 0)
    def _():
        pltpu.sync_copy(lbuf.at[pl.ds(0, CHUNK)],
                        lists_hbm.at[pl.ds(pl.multiple_of(
                            wid * PTS_PAD + flushed, CHUNK), CHUNK)])

    count = flushed + cursor

    sub_hbase = wid * (NSUB * SUBCAP)
    nch1 = lax.div(count + B1B_CH - 1, B1B_CH)

    def p1b_chunk(ci, carry):
        pltpu.sync_copy(lists_hbm.at[pl.ds(pl.multiple_of(
            wid * PTS_PAD + ci * B1B_CH, B1B_CH), B1B_CH)],
            lbuf.at[pl.ds(0, B1B_CH)])
        n_c = jnp.minimum(count - ci * B1B_CH, B1B_CH)
        nvec = lax.div(n_c + 15, 16)

        def vec_body(i, cy):
            w16 = lbuf[pl.ds(i * 16, 16)]
            local = jnp.bitwise_and(w16, CPW - 1)
            sub = lax.shift_right_logical(local, 9)
            valid = (i * 16 + lane) < n_c
            out = []
            for s in range(NSUB):
                ms = valid & (sub == s)
                mi = ms.astype(jnp.int32)
                off = cy[s] + plsc.cumsum(mi) - mi
                plsc.store_scatter(stg[s], [off], w16, mask=ms)
                out.append(cy[s] + jnp.sum(mi))
            return tuple(out) + cy[NSUB:]

        cy = lax.fori_loop(0, nvec, vec_body, carry)

        out = list(cy)
        for s in range(NSUB):
            def do_flush(cf, s=s):
                cur, fl = cf
                pltpu.sync_copy(
                    stg[s].at[pl.ds(0, B1B_CH)],
                    subs_hbm.at[pl.ds(pl.multiple_of(
                        sub_hbase + s * SUBCAP + fl, B1B_CH), B1B_CH)])

                def mv(i, _, s=s):
                    stg[s][pl.ds(i * 16, 16)] = \
                        stg[s][pl.ds(B1B_CH + i * 16, 16)]
                    return 0

                lax.fori_loop(0, B1B_CH // 16, mv, 0)
                return cur - B1B_CH, fl + B1B_CH

            cur, fl = lax.cond(out[s] >= B1B_CH, do_flush,
                               lambda cf: cf, (out[s], out[NSUB + s]))
            out[s], out[NSUB + s] = cur, fl
        return tuple(out)

    z = jnp.int32(0)
    carry = lax.fori_loop(0, nch1, p1b_chunk, (z,) * (2 * NSUB))
    sub_counts = []
    for s in range(NSUB):
        scur, sfl = carry[s], carry[NSUB + s]

        @pl.when(scur > 0)
        def _(s=s, sfl=sfl):
            pltpu.sync_copy(
                stg[s].at[pl.ds(0, B1B_CH)],
                subs_hbm.at[pl.ds(pl.multiple_of(
                    sub_hbase + s * SUBCAP + sfl, B1B_CH), B1B_CH)])

        sub_counts.append(scur + sfl)

    for s in range(NSUB):
        plsc.store_scatter(cbuf, [jnp.full((16,), s, jnp.int32)],
                           jnp.full((16,), sub_counts[s], jnp.int32),
                           mask=(lane == 0))

    neg = jnp.full((16,), NEG, jnp.float32)
    cntv = cbuf[pl.ds(0, 16)]
    ibufs = [ibuf0, ibuf1, ibuf2, ibuf3]
    rbufs = [rbuf0, rbuf1, rbuf2, rbuf3]
    sems = [sem0, sem1, sem2, sem3]

    def pass_body(ci, _):
        def sub_body(s, _):
            cbase = base_cell + s * CPS
            cnt = lax.gather(
                cntv, jnp.full((16, 1), s, jnp.int32),
                lax.GatherDimensionNumbers(
                    offset_dims=(), collapsed_slice_dims=(0,),
                    start_index_map=(0,)),
                slice_sizes=(1,),
                mode=lax.GatherScatterMode.PROMISE_IN_BOUNDS)[0]
            nmc = lax.div(cnt + MCH - 1, MCH)

            def prep_issue(mc, q):
                pltpu.sync_copy(
                    subs_hbm.at[pl.ds(pl.multiple_of(
                        sub_hbase + s * SUBCAP + mc * MCH, MCH), MCH)],
                    lbuf.at[pl.ds(q * MCH, MCH)])

                def idx_body(i, _):
                    w16 = lbuf[pl.ds(q * MCH + i * 16, 16)]
                    pid = lax.shift_right_logical(w16, 12)
                    pid = jnp.clip(pid, 0, NPTS - 1)
                    ibufs[q][pl.ds(i * 16, 16)] = ci * NPTS + pid
                    return 0

                lax.fori_loop(0, MCH // 16, idx_body, 0)
                pltpu.async_copy(h4v_hbm.at[ibufs[q]], rbufs[q], sems[q])

            def merge(mc, q):
                pltpu.make_async_copy(h4v_hbm.at[ibufs[q]], rbufs[q],
                                      sems[q]).wait()
                n_c = jnp.minimum(cnt - mc * MCH, MCH)

                def mg(i, _):
                    wv = lbuf[pl.ds(q * MCH + i * 16, 16)]
                    for k in range(16):
                        j = i * 16 + k
                        cl = jnp.where(j < n_c,
                                       jnp.bitwise_and(wv[k], CPS - 1),
                                       CPS)
                        for f in range(8):
                            row = rbufs[q][j, pl.ds(f * 16, 16)]
                            a = acc[cl, pl.ds(f * 16, 16)]
                            acc[cl, pl.ds(f * 16, 16)] = \
                                jnp.maximum(a, row)
                    return 0

                lax.fori_loop(0, lax.div(n_c + 15, 16), mg, 0)

            def init2(i, _):
                for f in range(8):
                    acc[i, pl.ds(f * 16, 16)] = neg
                return 0

            lax.fori_loop(0, CPS, init2, 0)

            for q in range(3):
                @pl.when(q < nmc)
                def _(q=q):
                    prep_issue(q, q)

            def quad_body(pi, _):
                for q in range(4):
                    c = pi * 4 + q

                    @pl.when(c < nmc)
                    def _(c=c, q=q):
                        @pl.when(c + 3 < nmc)
                        def _():
                            prep_issue(c + 3, (q + 3) % 4)

                        merge(c, q)

                return 0

            lax.fori_loop(0, lax.div(nmc + 3, 4), quad_body, 0)

            pltpu.sync_copy(
                acc.at[pl.ds(0, CPS)],
                grid_hbm.at[ci, pl.ds(pl.multiple_of(cbase, CPS), CPS)])
            return 0

        lax.fori_loop(0, NSUB, sub_body, 0)
        return 0

    lax.fori_loop(0, NCI, pass_body, 0)


def _scatter_max(cells_b, h4_b):
    h4v = h4_b.reshape(NCI * NPTS, 128)
    mesh = plsc.VectorSubcoreMesh(core_axis_name="c", subcore_axis_name="s")
    grid_out, _, _ = pl.kernel(
        _scatter_body,
        out_type=(jax.ShapeDtypeStruct((NCI, NCP, 128), jnp.float32),
                  jax.ShapeDtypeStruct((NW * PTS_PAD,), jnp.int32),
                  jax.ShapeDtypeStruct((NW * NSUB * SUBCAP,), jnp.int32)),
        mesh=mesh,
        compiler_params=pltpu.CompilerParams(needs_layout_passes=False),
        scratch_types=(
            [pltpu.VMEM((CHUNK,), jnp.int32),
             pltpu.VMEM((2 * CHUNK,), jnp.int32),
             pltpu.VMEM((MCH,), jnp.int32),
             pltpu.VMEM((MCH,), jnp.int32),
             pltpu.VMEM((MCH,), jnp.int32),
             pltpu.VMEM((MCH,), jnp.int32),
             pltpu.VMEM((MCH, 128), jnp.float32),
             pltpu.VMEM((MCH, 128), jnp.float32),
             pltpu.VMEM((MCH, 128), jnp.float32),
             pltpu.VMEM((MCH, 128), jnp.float32),
             pltpu.VMEM((CPS + 8, 128), jnp.float32)]
            + [pltpu.VMEM((B1B_STG,), jnp.int32)] * NSUB
            + [pltpu.SemaphoreType.DMA] * 4),
    )(cells_b, h4v)
    return grid_out


CB = 2048


def _final_kernel(g_ref, wc_ref, bc_ref, out_ref):
    g = jnp.concatenate([g_ref[0, ci] for ci in range(4)],
                        axis=1)
    present = (g[:, 0:1] > NEG)
    gm = jnp.where(present, g, 0.0)
    nhT = lax.dot_general(wc_ref[...], gm, (((0,), (1,)), ((), ())),
                          preferred_element_type=jnp.float32)
    nhT = jnp.maximum(nhT + bc_ref[0, 0][:, None], 0.0)
    out_ref[0] = jnp.where(present.T, nhT, 0.0)


def _final(grid2, Wc, bc):
    nb = NCP // CB
    return pl.pallas_call(
        _final_kernel, grid=(2, nb),
        in_specs=[
            pl.BlockSpec((1, NCI, CB, 128), lambda b, i: (b, 0, i, 0)),
            pl.BlockSpec((512, NH), lambda b, i: (0, 0)),
            pl.BlockSpec((1, 1, NH), lambda b, i: (0, 0, 0)),
        ],
        out_specs=pl.BlockSpec((1, NH, CB), lambda b, i: (b, 0, i)),
        out_shape=jax.ShapeDtypeStruct((2, NH, NCP), jnp.float32),
        compiler_params=pltpu.CompilerParams(
            dimension_semantics=("arbitrary", "arbitrary")),
    )(grid2, Wc, _vec(bc))


def kernel(pt_fea, xy_ind, circular_padding, bn0_g, bn0_b, W1, b1, bn1_g,
           bn1_b, W2, b2, bn2_g, bn2_b, W3, b3, bn3_g, bn3_b, W4, b4,
           Wc, bc):
    del circular_padding
    xp = jnp.pad(pt_fea, ((0, 0), (0, 0), (0, 1)))
    W1p = jnp.pad(W1, ((0, 1), (0, 0)))
    g0p = jnp.pad(bn0_g, (0, 1))
    b0p = jnp.pad(bn0_b, (0, 1))
    params = (g0p, b0p, W1p, b1, bn1_g, bn1_b, W2, b2, bn2_g, bn2_b,
              W3, b3, bn3_g, bn3_b, W4, b4)
    h4 = _mlp(xp, params)
    cells = xy_ind[..., 0] * G1 + xy_ind[..., 1]
    cells_p = jnp.pad(cells, ((0, 0), (0, PTS_PAD - NPTS)),
                      constant_values=NCP - 1)
    grid2 = jnp.stack([_scatter_max(cells_p[b], h4[b]) for b in range(2)])

    outc = _final(grid2, Wc, bc)[:, :, :NCELL]
    out = jnp.concatenate(
        [outc, jnp.zeros((2, NH, NSEG - NCELL), jnp.float32)], axis=2)
    return out.reshape(2, NH, G0, G1)


# speedup vs baseline: 3.1168x; 1.0570x over previous
import functools

import jax
import jax.numpy as jnp
from jax import lax
from jax.experimental import pallas as pl
from jax.experimental.pallas import tpu as pltpu
from jax.experimental.pallas import tpu_sc as plsc

G0, G1, NH = 480, 360, 32
NPTS = 100000
NCELL = G1 * G1
NCP = 131072
NSEG = G0 * G1
BLK = 4000
NBLK = NPTS // BLK
EPS = 1e-5
NEG = -1e30


def _stats_block(h):
    s = jnp.sum(h, axis=0)[None, None, :]
    sq = jnp.sum(h * h, axis=0)[None, None, :]
    return s, sq


def _p0_kernel(x_ref, s_ref, sq_ref):
    @pl.when(pl.program_id(1) == 0)
    def _():
        s_ref[...] = jnp.zeros_like(s_ref)
        sq_ref[...] = jnp.zeros_like(sq_ref)
    x = x_ref[0]
    s, sq = _stats_block(x)
    s_ref[...] += s
    sq_ref[...] += sq


def _layers(x, args, n):
    sc0, sh0 = args[0][0, 0], args[1][0, 0]
    h = x * sc0 + sh0
    for i in range(1, n + 1):
        W = args[2 + 4 * (i - 1)]
        b = args[3 + 4 * (i - 1)][0, 0]
        h = jnp.dot(h, W, preferred_element_type=jnp.float32) + b
        if i < n:
            sc = args[4 + 4 * (i - 1)][0, 0]
            sh = args[5 + 4 * (i - 1)][0, 0]
            h = jnp.maximum(h * sc + sh, 0.0)
    return h


def _stat_pass_kernel(n, *refs):
    x_ref = refs[0]
    args = [r[...] if r.shape[0] == 1 and len(r.shape) == 3 else r[...]
            for r in refs[1:-2]]
    s_ref, sq_ref = refs[-2], refs[-1]

    @pl.when(pl.program_id(1) == 0)
    def _():
        s_ref[...] = jnp.zeros_like(s_ref)
        sq_ref[...] = jnp.zeros_like(sq_ref)

    h = _layers(x_ref[0], args, n)
    s, sq = _stats_block(h)
    s_ref[...] += s
    sq_ref[...] += sq


def _p4_kernel(*refs):
    x_ref = refs[0]
    args = [r[...] for r in refs[1:-1]]
    out_ref = refs[-1]
    h = _layers(x_ref[0], args, 4)
    for ci in range(4):
        out_ref[ci] = h[:, ci * 128:(ci + 1) * 128]


def _vec(v):
    return v.reshape(1, 1, -1)


def _moment_specs(c):
    blk = pl.BlockSpec((1, 1, c), lambda b, i: (b, 0, 0))
    return [blk, blk]


def _vspec(c):
    return pl.BlockSpec((1, 1, c), lambda b, i: (0, 0, 0))


def _wspec(shape):
    return pl.BlockSpec(shape, lambda b, i: (0,) * len(shape))


def _scale_shift(s, sq, g, b):
    mean = s[0] / NPTS
    var = sq[0] / NPTS - mean * mean
    scale = g * lax.rsqrt(var + EPS)
    shift = b - mean * scale
    return scale, shift


def _mlp(xp, params):
    (bn0_g, bn0_b, W1, b1, bn1_g, bn1_b, W2, b2, bn2_g, bn2_b,
     W3, b3, bn3_g, bn3_b, W4, b4) = params
    grid = (2, NBLK)
    xspec = pl.BlockSpec((1, BLK, 8), lambda b, i: (b, i, 0))
    cp = pltpu.CompilerParams(
        dimension_semantics=("arbitrary", "arbitrary"))
    mom = lambda c: [jax.ShapeDtypeStruct((2, 1, c), jnp.float32)] * 2

    s0, sq0 = pl.pallas_call(
        _p0_kernel, grid=grid,
        in_specs=[xspec], out_specs=_moment_specs(8),
        out_shape=mom(8), compiler_params=cp)(xp)
    results = []
    for b in range(2):
        sc0, sh0 = _scale_shift(s0[b], sq0[b], bn0_g, bn0_b)
        args = [_vec(sc0), _vec(sh0), W1, _vec(b1)]
        specs = [_vspec(8), _vspec(8), _wspec(W1.shape), _vspec(64)]
        dims = [64, 128, 256]
        gs = [bn1_g, bn2_g, bn3_g]
        bs = [bn1_b, bn2_b, bn3_b]
        Ws = [W2, W3, W4]
        lbs = [b2, b3, b4]
        xb = xp[b:b + 1]
        gridb = (1, NBLK)
        for n in (1, 2, 3):
            c = dims[n - 1]
            s, sq = pl.pallas_call(
                functools.partial(_stat_pass_kernel, n), grid=gridb,
                in_specs=[xspec] + specs,
                out_specs=_moment_specs(c),
                out_shape=mom(c)[0:1] + mom(c)[0:1],
                compiler_params=cp)(xb, *args)
            sc, sh = _scale_shift(s[0], sq[0], gs[n - 1], bs[n - 1])
            W, lb = Ws[n - 1], lbs[n - 1]
            args += [_vec(sc), _vec(sh), W, _vec(lb)]
            specs += [_vspec(c), _vspec(c), _wspec(W.shape),
                      _vspec(W.shape[1])]
        h4 = pl.pallas_call(
            _p4_kernel, grid=gridb,
            in_specs=[xspec] + specs,
            out_specs=pl.BlockSpec((4, BLK, 128), lambda b, i: (0, i, 0)),
            out_shape=jax.ShapeDtypeStruct((4, NPTS, 128), jnp.float32),
            compiler_params=cp)(xb, *args)
        results.append(h4)
    return results


NW = 32
CPW = NCP // NW
NSUB = 8
CPS = CPW // NSUB
PTS_PAD = 102400
CHUNK = 2048
NCHUNK = PTS_PAD // CHUNK
SUBCAP = PTS_PAD + CHUNK
MCH = 64
B1B_CH = 1024
B1B_STG = 2048
NCI = 4


def _scatter_body(cells_hbm, h4v_hbm, grid_hbm, lists_hbm, subs_hbm,
                  cbuf, lbuf, ibuf0, ibuf1, ibuf2, ibuf3,
                  rbuf0, rbuf1, rbuf2, rbuf3, acc,
                  st0, st1, st2, st3, st4, st5, st6, st7,
                  sem0, sem1, sem2, sem3):
    stg = [st0, st1, st2, st3, st4, st5, st6, st7]
    wid = lax.axis_index("s") * 2 + lax.axis_index("c")
    base_cell = wid * CPW
    lane = lax.iota(jnp.int32, 16)

    def chunk_body(c, carry):
        cursor, flushed = carry
        pltpu.sync_copy(cells_hbm.at[pl.ds(c * CHUNK, CHUNK)], cbuf)

        def vec_body(i, cur):
            v = cbuf[pl.ds(i * 16, 16)]
            local = v - base_cell
            m = (local >= 0) & (local < CPW)
            mi = m.astype(jnp.int32)
            pid = c * CHUNK + i * 16 + lane
            packed = pid * CPW + local
            off = cur + plsc.cumsum(mi) - mi
            plsc.store_scatter(lbuf, [off], packed, mask=m)
            return cur + jnp.sum(mi)

        cursor = lax.fori_loop(0, CHUNK // 16, vec_body, cursor)

        def do_flush(cf):
            cur, fl = cf
            pltpu.sync_copy(lbuf.at[pl.ds(0, CHUNK)],
                            lists_hbm.at[pl.ds(pl.multiple_of(
                                wid * PTS_PAD + fl, CHUNK), CHUNK)])

            def mv(i, _):
                lbuf[pl.ds(i * 16, 16)] = lbuf[pl.ds(CHUNK + i * 16, 16)]
                return 0

            lax.fori_loop(0, CHUNK // 16, mv, 0)
            return cur - CHUNK, fl + CHUNK

        return lax.cond(cursor >= CHUNK, do_flush, lambda cf: cf,
                        (cursor, flushed))

    cursor, flushed = lax.fori_loop(0, NCHUNK, chunk_body,
                                    (jnp.int32(0), jnp.int32(0)))

    @pl.when(cursor > 0)
    def _():
        pltpu.sync_copy(lbuf.at[pl.ds(0, CHUNK)],
                        lists_hbm.at[pl.ds(pl.multiple_of(
                            wid * PTS_PAD + flushed, CHUNK), CHUNK)])

    count = flushed + cursor

    sub_hbase = wid * (NSUB * SUBCAP)
    nch1 = lax.div(count + B1B_CH - 1, B1B_CH)

    def p1b_chunk(ci, carry):
        pltpu.sync_copy(lists_hbm.at[pl.ds(pl.multiple_of(
            wid * PTS_PAD + ci * B1B_CH, B1B_CH), B1B_CH)],
            lbuf.at[pl.ds(0, B1B_CH)])
        n_c = jnp.minimum(count - ci * B1B_CH, B1B_CH)
        nvec = lax.div(n_c + 15, 16)

        def vec_body(i, cy):
            w16 = lbuf[pl.ds(i * 16, 16)]
            local = jnp.bitwise_and(w16, CPW - 1)
            sub = lax.shift_right_logical(local, 9)
            valid = (i * 16 + lane) < n_c
            out = []
            for s in range(NSUB):
                ms = valid & (sub == s)
                mi = ms.astype(jnp.int32)
                off = cy[s] + plsc.cumsum(mi) - mi
                plsc.store_scatter(stg[s], [off], w16, mask=ms)
                out.append(cy[s] + jnp.sum(mi))
            return tuple(out) + cy[NSUB:]

        cy = lax.fori_loop(0, nvec, vec_body, carry)

        out = list(cy)
        for s in range(NSUB):
            def do_flush(cf, s=s):
                cur, fl = cf
                pltpu.sync_copy(
                    stg[s].at[pl.ds(0, B1B_CH)],
                    subs_hbm.at[pl.ds(pl.multiple_of(
                        sub_hbase + s * SUBCAP + fl, B1B_CH), B1B_CH)])

                def mv(i, _, s=s):
                    stg[s][pl.ds(i * 16, 16)] = \
                        stg[s][pl.ds(B1B_CH + i * 16, 16)]
                    return 0

                lax.fori_loop(0, B1B_CH // 16, mv, 0)
                return cur - B1B_CH, fl + B1B_CH

            cur, fl = lax.cond(out[s] >= B1B_CH, do_flush,
                               lambda cf: cf, (out[s], out[NSUB + s]))
            out[s], out[NSUB + s] = cur, fl
        return tuple(out)

    z = jnp.int32(0)
    carry = lax.fori_loop(0, nch1, p1b_chunk, (z,) * (2 * NSUB))
    sub_counts = []
    for s in range(NSUB):
        scur, sfl = carry[s], carry[NSUB + s]

        @pl.when(scur > 0)
        def _(s=s, sfl=sfl):
            pltpu.sync_copy(
                stg[s].at[pl.ds(0, B1B_CH)],
                subs_hbm.at[pl.ds(pl.multiple_of(
                    sub_hbase + s * SUBCAP + sfl, B1B_CH), B1B_CH)])

        sub_counts.append(scur + sfl)

    for s in range(NSUB):
        plsc.store_scatter(cbuf, [jnp.full((16,), s, jnp.int32)],
                           jnp.full((16,), sub_counts[s], jnp.int32),
                           mask=(lane == 0))

    neg = jnp.full((16,), NEG, jnp.float32)
    cntv = cbuf[pl.ds(0, 16)]
    ibufs = [ibuf0, ibuf1, ibuf2, ibuf3]
    rbufs = [rbuf0, rbuf1, rbuf2, rbuf3]
    sems = [sem0, sem1, sem2, sem3]

    def pass_body(ci, _):
        def sub_body(s, _):
            cbase = base_cell + s * CPS
            cnt = lax.gather(
                cntv, jnp.full((16, 1), s, jnp.int32),
                lax.GatherDimensionNumbers(
                    offset_dims=(), collapsed_slice_dims=(0,),
                    start_index_map=(0,)),
                slice_sizes=(1,),
                mode=lax.GatherScatterMode.PROMISE_IN_BOUNDS)[0]
            nmc = lax.div(cnt + MCH - 1, MCH)

            def prep_issue(mc, q):
                pltpu.sync_copy(
                    subs_hbm.at[pl.ds(pl.multiple_of(
                        sub_hbase + s * SUBCAP + mc * MCH, MCH), MCH)],
                    lbuf.at[pl.ds(q * MCH, MCH)])

                def idx_body(i, _):
                    w16 = lbuf[pl.ds(q * MCH + i * 16, 16)]
                    pid = lax.shift_right_logical(w16, 12)
                    pid = jnp.clip(pid, 0, NPTS - 1)
                    ibufs[q][pl.ds(i * 16, 16)] = ci * NPTS + pid
                    return 0

                lax.fori_loop(0, MCH // 16, idx_body, 0)
                pltpu.async_copy(h4v_hbm.at[ibufs[q]], rbufs[q], sems[q])

            def merge(mc, q):
                pltpu.make_async_copy(h4v_hbm.at[ibufs[q]], rbufs[q],
                                      sems[q]).wait()
                n_c = jnp.minimum(cnt - mc * MCH, MCH)

                def mg(i, _):
                    wv = lbuf[pl.ds(q * MCH + i * 16, 16)]
                    for k in range(16):
                        j = i * 16 + k
                        cl = jnp.where(j < n_c,
                                       jnp.bitwise_and(wv[k], CPS - 1),
                                       CPS)
                        for f in range(8):
                            row = rbufs[q][j, pl.ds(f * 16, 16)]
                            a = acc[cl, pl.ds(f * 16, 16)]
                            acc[cl, pl.ds(f * 16, 16)] = \
                                jnp.maximum(a, row)
                    return 0

                lax.fori_loop(0, lax.div(n_c + 15, 16), mg, 0)

            def init2(i, _):
                for f in range(8):
                    acc[i, pl.ds(f * 16, 16)] = neg
                return 0

            lax.fori_loop(0, CPS, init2, 0)

            for q in range(3):
                @pl.when(q < nmc)
                def _(q=q):
                    prep_issue(q, q)

            def quad_body(pi, _):
                for q in range(4):
                    c = pi * 4 + q

                    @pl.when(c < nmc)
                    def _(c=c, q=q):
                        @pl.when(c + 3 < nmc)
                        def _():
                            prep_issue(c + 3, (q + 3) % 4)

                        merge(c, q)

                return 0

            lax.fori_loop(0, lax.div(nmc + 3, 4), quad_body, 0)

            pltpu.sync_copy(
                acc.at[pl.ds(0, CPS)],
                grid_hbm.at[ci, pl.ds(pl.multiple_of(cbase, CPS), CPS)])
            return 0

        lax.fori_loop(0, NSUB, sub_body, 0)
        return 0

    lax.fori_loop(0, NCI, pass_body, 0)


def _scatter_max(cells_b, h4_b):
    h4v = h4_b.reshape(NCI * NPTS, 128)
    mesh = plsc.VectorSubcoreMesh(core_axis_name="c", subcore_axis_name="s")
    grid_out, _, _ = pl.kernel(
        _scatter_body,
        out_type=(jax.ShapeDtypeStruct((NCI, NCP, 128), jnp.float32),
                  jax.ShapeDtypeStruct((NW * PTS_PAD,), jnp.int32),
                  jax.ShapeDtypeStruct((NW * NSUB * SUBCAP,), jnp.int32)),
        mesh=mesh,
        compiler_params=pltpu.CompilerParams(needs_layout_passes=False),
        scratch_types=(
            [pltpu.VMEM((CHUNK,), jnp.int32),
             pltpu.VMEM((2 * CHUNK,), jnp.int32),
             pltpu.VMEM((MCH,), jnp.int32),
             pltpu.VMEM((MCH,), jnp.int32),
             pltpu.VMEM((MCH,), jnp.int32),
             pltpu.VMEM((MCH,), jnp.int32),
             pltpu.VMEM((MCH, 128), jnp.float32),
             pltpu.VMEM((MCH, 128), jnp.float32),
             pltpu.VMEM((MCH, 128), jnp.float32),
             pltpu.VMEM((MCH, 128), jnp.float32),
             pltpu.VMEM((CPS + 8, 128), jnp.float32)]
            + [pltpu.VMEM((B1B_STG,), jnp.int32)] * NSUB
            + [pltpu.SemaphoreType.DMA] * 4),
    )(cells_b, h4v)
    return grid_out


CB = 2048


def _final_kernel(g_ref, wc_ref, bc_ref, out_ref):
    g = jnp.concatenate([g_ref[0, ci] for ci in range(4)],
                        axis=1)
    present = (g[:, 0:1] > NEG)
    gm = jnp.where(present, g, 0.0)
    nhT = lax.dot_general(wc_ref[...], gm, (((0,), (1,)), ((), ())),
                          preferred_element_type=jnp.float32)
    nhT = jnp.maximum(nhT + bc_ref[0, 0][:, None], 0.0)
    out_ref[0] = jnp.where(present.T, nhT, 0.0)


def _final(grid2, Wc, bc):
    nb = NCP // CB
    return pl.pallas_call(
        _final_kernel, grid=(2, nb),
        in_specs=[
            pl.BlockSpec((1, NCI, CB, 128), lambda b, i: (b, 0, i, 0)),
            pl.BlockSpec((512, NH), lambda b, i: (0, 0)),
            pl.BlockSpec((1, 1, NH), lambda b, i: (0, 0, 0)),
        ],
        out_specs=pl.BlockSpec((1, NH, CB), lambda b, i: (b, 0, i)),
        out_shape=jax.ShapeDtypeStruct((2, NH, NCP), jnp.float32),
        compiler_params=pltpu.CompilerParams(
            dimension_semantics=("arbitrary", "arbitrary")),
    )(grid2, Wc, _vec(bc))


def kernel(pt_fea, xy_ind, circular_padding, bn0_g, bn0_b, W1, b1, bn1_g,
           bn1_b, W2, b2, bn2_g, bn2_b, W3, b3, bn3_g, bn3_b, W4, b4,
           Wc, bc):
    del circular_padding
    xp = jnp.pad(pt_fea, ((0, 0), (0, 0), (0, 1)))
    W1p = jnp.pad(W1, ((0, 1), (0, 0)))
    g0p = jnp.pad(bn0_g, (0, 1))
    b0p = jnp.pad(bn0_b, (0, 1))
    params = (g0p, b0p, W1p, b1, bn1_g, bn1_b, W2, b2, bn2_g, bn2_b,
              W3, b3, bn3_g, bn3_b, W4, b4)
    h4 = _mlp(xp, params)
    cells = xy_ind[..., 0] * G1 + xy_ind[..., 1]
    cells_p = jnp.pad(cells, ((0, 0), (0, PTS_PAD - NPTS)),
                      constant_values=NCP - 1)
    grid2 = jnp.stack([_scatter_max(cells_p[b], h4[b]) for b in range(2)])

    outc = _final(grid2, Wc, bc)[:, :, :NCELL]
    out = jnp.concatenate(
        [outc, jnp.zeros((2, NH, NSEG - NCELL), jnp.float32)], axis=2)
    return out.reshape(2, NH, G0, G1)
